# Initial kernel scaffold; baseline (speedup 1.0000x reference)
import jax
import jax.numpy as jnp
from jax.experimental import pallas as pl


def kernel(x, edge_index, batch, W1, b1, W2, b2, Wl1, bl1, Wl2, bl2):
    raise NotImplementedError("write your pallas kernel here")


# speedup vs baseline: 22.1277x; 22.1277x over previous
import jax
import jax.numpy as jnp
from jax import lax
from jax.experimental import pallas as pl
from jax.experimental.pallas import tpu as pltpu
from jax.experimental.pallas import tpu_sc as plsc

N = 10000
E = 320000
D = 128
B = 64

NC = 2
NS = 16
NW = NC * NS
EPW = E // NW
K = 125
NCHUNK = EPW // K
DEGW = 16
FT = 10
FR = N // FT
ZRA = 40

_sc_mesh = plsc.VectorSubcoreMesh(core_axis_name="c", subcore_axis_name="s")


def _deg_body(dst_hbm, ones_hbm, zeros_hbm, deg_hbm, dstv, onesv, degsh):
    cid = lax.axis_index("c")
    sid = lax.axis_index("s")
    wid = cid * NS + sid

    @pl.when(sid < FT)
    def _():
        pltpu.sync_copy(zeros_hbm, degsh.at[pl.ds(sid * FR, FR)])

    pltpu.sync_copy(dst_hbm.at[wid], dstv)
    pltpu.sync_copy(ones_hbm, onesv)
    plsc.subcore_barrier()

    def body(j, c):
        pltpu.sync_copy(onesv, degsh.at[dstv.at[j]], add=True)
        return c

    lax.fori_loop(0, NCHUNK, body, 0)
    plsc.subcore_barrier()

    @pl.when(sid < FT)
    def _():
        sl = pl.ds(sid * FR, FR)
        pltpu.sync_copy(degsh.at[sl], deg_hbm.at[cid, sl])


_deg_call = pl.kernel(
    _deg_body,
    out_type=jax.ShapeDtypeStruct((NC, N, DEGW), jnp.float32),
    mesh=_sc_mesh,
    scratch_types=[
        pltpu.VMEM((NCHUNK, K), jnp.int32),
        pltpu.VMEM((K, DEGW), jnp.float32),
        pltpu.VMEM_SHARED((N, DEGW), jnp.float32),
    ],
)


def _agg_body(y_hbm, src_hbm, dst_hbm, zeros_hbm, agg_hbm,
              srcv, dstv, buf, zbuf, aggsh):
    cid = lax.axis_index("c")
    sid = lax.axis_index("s")
    wid = cid * NS + sid

    @pl.when(sid < FT)
    def _():
        pltpu.sync_copy(zeros_hbm, zbuf)
        for t in range(FR // ZRA):
            pltpu.sync_copy(zbuf, aggsh.at[pl.ds(sid * FR + t * ZRA, ZRA)])

    pltpu.sync_copy(src_hbm.at[wid], srcv)
    pltpu.sync_copy(dst_hbm.at[wid], dstv)
    plsc.subcore_barrier()

    def body(j, c):
        pltpu.sync_copy(y_hbm.at[srcv.at[j]], buf)
        pltpu.sync_copy(buf, aggsh.at[dstv.at[j]], add=True)
        return c

    lax.fori_loop(0, NCHUNK, body, 0)
    plsc.subcore_barrier()

    @pl.when(sid < FT)
    def _():
        sl = pl.ds(sid * FR, FR)
        pltpu.sync_copy(aggsh.at[sl], agg_hbm.at[cid, sl])


_agg_call = pl.kernel(
    _agg_body,
    out_type=jax.ShapeDtypeStruct((NC, N, D), jnp.float32),
    mesh=_sc_mesh,
    scratch_types=[
        pltpu.VMEM((NCHUNK, K), jnp.int32),
        pltpu.VMEM((NCHUNK, K), jnp.int32),
        pltpu.VMEM((K, D), jnp.float32),
        pltpu.VMEM((ZRA, D), jnp.float32),
        pltpu.VMEM_SHARED((N, D), jnp.float32),
    ],
)


_R = 1000
_G = N // _R


def _dinv_of(deg_ref):
    return lax.rsqrt(jnp.sum(deg_ref[...], axis=(0, 2)) + 1.0)[:, None]


def _y1_body(deg_ref, x_ref, w_ref, y_ref):
    xw = jnp.dot(x_ref[...], w_ref[...], preferred_element_type=jnp.float32)
    y_ref[...] = xw * _dinv_of(deg_ref)


def _y1_call(deg, x, w1):
    return pl.pallas_call(
        _y1_body,
        grid=(_G,),
        in_specs=[
            pl.BlockSpec((NC, _R, DEGW), lambda i: (0, i, 0)),
            pl.BlockSpec((_R, D), lambda i: (i, 0)),
            pl.BlockSpec((D, D), lambda i: (0, 0)),
        ],
        out_specs=pl.BlockSpec((_R, D), lambda i: (i, 0)),
        out_shape=jax.ShapeDtypeStruct((N, D), jnp.float32),
    )(deg, x, w1)


def _y2_body(agg_ref, y1_ref, deg_ref, b1_ref, w2_ref, y2_ref):
    dinv = _dinv_of(deg_ref)
    a = agg_ref[0] + agg_ref[1] + y1_ref[...]
    h = jnp.maximum(a * dinv + b1_ref[...], 0.0)
    y2_ref[...] = jnp.dot(h, w2_ref[...],
                          preferred_element_type=jnp.float32) * dinv


def _y2_call(agg, y1, deg, b1, w2):
    return pl.pallas_call(
        _y2_body,
        grid=(_G,),
        in_specs=[
            pl.BlockSpec((NC, _R, D), lambda i: (0, i, 0)),
            pl.BlockSpec((_R, D), lambda i: (i, 0)),
            pl.BlockSpec((NC, _R, DEGW), lambda i: (0, i, 0)),
            pl.BlockSpec((D,), lambda i: (0,)),
            pl.BlockSpec((D, D), lambda i: (0, 0)),
        ],
        out_specs=pl.BlockSpec((_R, D), lambda i: (i, 0)),
        out_shape=jax.ShapeDtypeStruct((N, D), jnp.float32),
    )(agg, y1, deg, b1, w2)


def _final_body(agg_ref, y2_ref, deg_ref, b2_ref, batch_ref,
                wl1_ref, bl1_ref, wl2_ref, bl2_ref, out_ref, gsum, cnt):
    i = pl.program_id(0)

    @pl.when(i == 0)
    def _():
        gsum[...] = jnp.zeros_like(gsum)
        cnt[...] = jnp.zeros_like(cnt)

    dinv = _dinv_of(deg_ref)
    a = agg_ref[0] + agg_ref[1] + y2_ref[...]
    h = jnp.maximum(a * dinv + b2_ref[...], 0.0)
    onehot = (batch_ref[0, 0][:, None]
              == lax.broadcasted_iota(jnp.int32, (1, B), 1)
              ).astype(jnp.float32)
    gsum[...] += lax.dot_general(onehot, h, (((0,), (0,)), ((), ())),
                                 preferred_element_type=jnp.float32)
    cnt[...] += jnp.sum(onehot, axis=0)[None, :]

    @pl.when(i == pl.num_programs(0) - 1)
    def _():
        g = gsum[...] / jnp.maximum(cnt[...], 1.0)[0][:, None]
        g1 = jnp.maximum(
            jnp.dot(g, wl1_ref[...], preferred_element_type=jnp.float32)
            + bl1_ref[...], 0.0)
        out_ref[...] = (jnp.dot(g1, wl2_ref[...],
                                preferred_element_type=jnp.float32)
                        + bl2_ref[...])


def _final_call(agg, y2, deg, b2, batch, wl1, bl1, wl2, bl2):
    batch3 = batch.reshape(_G, 1, _R)
    return pl.pallas_call(
        _final_body,
        grid=(_G,),
        in_specs=[
            pl.BlockSpec((NC, _R, D), lambda i: (0, i, 0)),
            pl.BlockSpec((_R, D), lambda i: (i, 0)),
            pl.BlockSpec((NC, _R, DEGW), lambda i: (0, i, 0)),
            pl.BlockSpec((D,), lambda i: (0,)),
            pl.BlockSpec((1, 1, _R), lambda i: (i, 0, 0)),
            pl.BlockSpec((D, D), lambda i: (0, 0)),
            pl.BlockSpec((D,), lambda i: (0,)),
            pl.BlockSpec((D, D), lambda i: (0, 0)),
            pl.BlockSpec((D,), lambda i: (0,)),
        ],
        out_specs=pl.BlockSpec((B, D), lambda i: (0, 0)),
        out_shape=jax.ShapeDtypeStruct((B, D), jnp.float32),
        scratch_shapes=[
            pltpu.VMEM((B, D), jnp.float32),
            pltpu.VMEM((1, B), jnp.float32),
        ],
    )(agg, y2, deg, b2, batch3, wl1, bl1, wl2, bl2)


def kernel(x, edge_index, batch, W1, b1, W2, b2, Wl1, bl1, Wl2, bl2):
    src = edge_index[0].reshape(NW, NCHUNK, K)
    dst = edge_index[1].reshape(NW, NCHUNK, K)
    ones = jnp.ones((K, DEGW), jnp.float32)
    zeros_deg = jnp.zeros((FR, DEGW), jnp.float32)
    zeros_agg = jnp.zeros((ZRA, D), jnp.float32)

    deg = _deg_call(dst, ones, zeros_deg)
    y1 = _y1_call(deg, x, W1)
    agg1 = _agg_call(y1, src, dst, zeros_agg)
    y2 = _y2_call(agg1, y1, deg, b1, W2)
    agg2 = _agg_call(y2, src, dst, zeros_agg)
    return _final_call(agg2, y2, deg, b2, batch, Wl1, bl1, Wl2, bl2)
